# baseline (speedup 1.0000x reference)
import functools

import jax
import jax.numpy as jnp
from jax import lax
from jax.experimental import pallas as pl
from jax.experimental.pallas import tpu as pltpu
from jax.experimental.pallas import tpu_sc as plsc

_EPS = 1e-6
_NC, _NS = 2, 16
_NW = _NC * _NS
_CHUNK = 64
_PK = 128


def _sc_gather(u4, u2, p4, n4f, utp, uup, vp):
    B = u4.shape[0]
    NNEG = n4f.shape[0] // B
    rows_per_w = B // _NW
    n_chunks = rows_per_w // _CHUNK

    mesh = plsc.VectorSubcoreMesh(core_axis_name="c", subcore_axis_name="s")

    @functools.partial(
        pl.kernel,
        out_type=tuple(
            jax.ShapeDtypeStruct((B, _PK), jnp.float32) for _ in range(3 + NNEG)
        ),
        mesh=mesh,
        scratch_types=(
            pltpu.VMEM((_CHUNK,), jnp.int32),
            pltpu.VMEM((_CHUNK,), jnp.int32),
            pltpu.VMEM((_CHUNK,), jnp.int32),
            pltpu.VMEM((NNEG, _CHUNK), jnp.int32),
            pltpu.VMEM((3 + NNEG, _CHUNK, _PK), jnp.float32),
            pltpu.SemaphoreType.DMA,
        ),
        compiler_params=pltpu.CompilerParams(use_tc_tiling_on_sc=True),
    )
    def k(u4_h, u2_h, p4_h, n4_h, ut_tab, uu_tab, v_tab,
          *out_and_scratch):
        outs = out_and_scratch[:3 + NNEG]
        uix, wix, pix, nix, buf, sem = out_and_scratch[3 + NNEG:]
        wid = lax.axis_index("s") * _NC + lax.axis_index("c")
        for ci in range(n_chunks):
            base = wid * rows_per_w + ci * _CHUNK
            rows = pl.ds(base, _CHUNK)
            pltpu.sync_copy(u4_h.at[rows], uix)
            pltpu.sync_copy(u2_h.at[rows], wix)
            pltpu.sync_copy(p4_h.at[rows], pix)
            for kn in range(NNEG):
                pltpu.sync_copy(n4_h.at[pl.ds(kn * B + base, _CHUNK)],
                                nix.at[kn])
            cps = [
                pltpu.async_copy(ut_tab.at[uix], buf.at[0], sem),
                pltpu.async_copy(uu_tab.at[wix], buf.at[1], sem),
                pltpu.async_copy(v_tab.at[pix], buf.at[2], sem),
            ]
            cps += [
                pltpu.async_copy(v_tab.at[nix.at[kn]], buf.at[3 + kn], sem)
                for kn in range(NNEG)
            ]
            for cp in cps:
                cp.wait()
            for j in range(3 + NNEG):
                pltpu.sync_copy(buf.at[j], outs[j].at[rows])

    return k(u4, u2, p4, n4f, utp, uup, vp)


def _sel4(x, rem, D):
    acc = None
    for m in range(4):
        part = jnp.where(rem == m, x[:, m * D:(m + 1) * D], 0.0)
        acc = part if acc is None else acc + part
    return acc


def _tc_loss(gs, ur4, ur2, pr4, nr4, das2, wt, dab):
    B = gs[0].shape[0]
    D = dab.shape[1]
    DAP = dab.shape[0]
    NNEG = len(gs) - 3
    BLK = 1024
    grid = B // BLK

    def body(ut_ref, uu_ref, pi_ref, n0, n1, n2, n3, n4,
             ur4_ref, ur2_ref, pr4_ref, nr4_ref, das_ref, w_ref, dab_ref,
             out_ref):
        ut = _sel4(ut_ref[...], ur4_ref[...], D)
        uu = uu_ref[...]
        uu64 = jnp.where(ur2_ref[...] == 0, uu[:, :2 * D], uu[:, 2 * D:])
        up = ut * 2.0 + uu64[:, :D] + uu64[:, D:]
        onehot = (
            lax.broadcasted_iota(jnp.int32, (BLK, DAP), 1) == das_ref[...]
        ).astype(jnp.float32)
        u = jnp.dot(up, w_ref[...], preferred_element_type=jnp.float32)
        u = u + jnp.dot(onehot, dab_ref[...],
                        preferred_element_type=jnp.float32)
        pi = _sel4(pi_ref[...], pr4_ref[...], D)
        dpos = u - pi + _EPS
        dp = jnp.sqrt(jnp.sum(dpos * dpos, axis=1))
        acc = jnp.zeros((), jnp.float32)
        nrefs = [n0, n1, n2, n3, n4]
        for kn in range(NNEG):
            xj = _sel4(nrefs[kn][...], nr4_ref[..., kn:kn + 1], D)
            dneg = u - xj + _EPS
            dn = jnp.sqrt(jnp.sum(dneg * dneg, axis=1))
            acc = acc + jnp.sum(jnp.maximum(dp - dn + 1.0, 0.0))

        @pl.when(pl.program_id(0) == 0)
        def _():
            out_ref[...] = jnp.zeros_like(out_ref)

        out_ref[...] += (acc * (1.0 / B)).reshape(1, 1)

    out = pl.pallas_call(
        body,
        grid=(grid,),
        in_specs=(
            [pl.BlockSpec((BLK, _PK), lambda i: (i, 0)) for _ in range(8)]
            + [pl.BlockSpec((BLK, 1), lambda i: (i, 0)) for _ in range(3)]
            + [pl.BlockSpec((BLK, 5), lambda i: (i, 0)),
               pl.BlockSpec((BLK, 1), lambda i: (i, 0)),
               pl.BlockSpec((D, D), lambda i: (0, 0)),
               pl.BlockSpec((DAP, D), lambda i: (0, 0))]
        ),
        out_specs=pl.BlockSpec((1, 1), lambda i: (0, 0)),
        out_shape=jax.ShapeDtypeStruct((1, 1), jnp.float32),
    )(*gs, ur4, ur2, pr4, nr4, das2, wt, dab)
    return out[0, 0]


def kernel(phase, users, pos_job_ids, behavior_ids, das, neg_job_id_lists,
           U_true, U, V, da_tab, W, b):
    del phase, behavior_ids
    NUSER, BEHm1, D = U.shape
    NJOB = V.shape[0]
    B = users.shape[0]
    DA = da_tab.shape[0] - 1
    vp = V.reshape(NJOB // 4, 4 * D)
    uup = U.reshape(NUSER // 2, 2 * BEHm1 * D)
    utp = U_true[:NUSER].reshape(NUSER // 4, 4 * D)
    negf = neg_job_id_lists.T.reshape(-1)
    u4 = (users // 4).astype(jnp.int32)
    u2 = (users // 2).astype(jnp.int32)
    p4 = (pos_job_ids // 4).astype(jnp.int32)
    n4f = (negf // 4).astype(jnp.int32)
    ur4 = (users % 4).astype(jnp.int32).reshape(B, 1)
    ur2 = (users % 2).astype(jnp.int32).reshape(B, 1)
    pr4 = (pos_job_ids % 4).astype(jnp.int32).reshape(B, 1)
    nr4 = (neg_job_id_lists % 4).astype(jnp.int32)
    das_c = jnp.clip(das, 0, DA).astype(jnp.int32).reshape(B, 1)
    DAP = 128
    dab = jnp.zeros((DAP, D), jnp.float32).at[:DA + 1].set(da_tab + b[None, :])
    gs = _sc_gather(u4, u2, p4, n4f, utp, uup, vp)
    return _tc_loss(gs, ur4, ur2, pr4, nr4, das_c, W.T, dab)


# speedup vs baseline: 1.4012x; 1.4012x over previous
import functools

import jax
import jax.numpy as jnp
from jax import lax
from jax.experimental import pallas as pl
from jax.experimental.pallas import tpu as pltpu
from jax.experimental.pallas import tpu_sc as plsc

_EPS = 1e-6
_NC, _NS = 2, 16
_NW = _NC * _NS
_CHUNK = 128
_L = 16
_PK = 128


def _sc_gather(users, pos, negf, u_true_s, u3, v):
    B = users.shape[0]
    D = u_true_s.shape[1]
    NNEG = negf.shape[0] // B
    rows_per_w = B // _NW
    n_chunks = rows_per_w // _CHUNK

    mesh = plsc.VectorSubcoreMesh(core_axis_name="c", subcore_axis_name="s")

    @functools.partial(
        pl.kernel,
        out_type=(
            jax.ShapeDtypeStruct((B, _PK), jnp.float32),
            jax.ShapeDtypeStruct((B, _PK), jnp.float32),
        ),
        mesh=mesh,
        scratch_types=(
            pltpu.VMEM((_CHUNK,), jnp.int32),
            pltpu.VMEM((_CHUNK,), jnp.int32),
            pltpu.VMEM((NNEG, _CHUNK), jnp.int32),
            pltpu.VMEM((_CHUNK, D), jnp.float32),
            pltpu.VMEM((_CHUNK, 2 * D), jnp.float32),
            pltpu.VMEM((_CHUNK, D), jnp.float32),
            pltpu.VMEM((_CHUNK, D), jnp.float32),
            pltpu.VMEM((NNEG, _CHUNK, D), jnp.float32),
            pltpu.SemaphoreType.DMA,
        ),
        compiler_params=pltpu.CompilerParams(use_tc_tiling_on_sc=False),
    )
    def k(users_h, pos_h, neg_h, ut_tab, u3_tab, v_tab,
          o1, o2, uix, pix, nix, ut_b, uu_b, up_b, pi_b, nj_b, sem):
        wid = lax.axis_index("s") * _NC + lax.axis_index("c")
        for ci in range(n_chunks):
            base = wid * rows_per_w + ci * _CHUNK
            rows = pl.ds(base, _CHUNK)
            pltpu.sync_copy(users_h.at[rows], uix)
            pltpu.sync_copy(pos_h.at[rows], pix)
            for kn in range(NNEG):
                pltpu.sync_copy(neg_h.at[pl.ds(kn * B + base, _CHUNK)],
                                nix.at[kn])
            cps = [
                pltpu.async_copy(ut_tab.at[uix], ut_b, sem),
                pltpu.async_copy(u3_tab.at[uix], uu_b, sem),
                pltpu.async_copy(v_tab.at[pix], pi_b, sem),
            ]
            cps += [
                pltpu.async_copy(v_tab.at[nix.at[kn]], nj_b.at[kn], sem)
                for kn in range(NNEG)
            ]
            cps[0].wait()
            cps[1].wait()

            @pl.loop(0, _CHUNK)
            def _(r):
                for h in range(D // _L):
                    sl = pl.ds(h * _L, _L)
                    up_b[r, sl] = (ut_b[r, sl] * 2.0 + uu_b[r, sl]
                                   + uu_b[r, pl.ds(D + h * _L, _L)])

            for cp in cps[2:]:
                cp.wait()
            pltpu.sync_copy(up_b, o1.at[rows, pl.ds(0, D)])
            pltpu.sync_copy(pi_b, o1.at[rows, pl.ds(D, D)])
            pltpu.sync_copy(nj_b.at[0], o1.at[rows, pl.ds(2 * D, D)])
            pltpu.sync_copy(nj_b.at[1], o1.at[rows, pl.ds(3 * D, D)])
            pltpu.sync_copy(nj_b.at[2], o2.at[rows, pl.ds(0, D)])
            pltpu.sync_copy(nj_b.at[3], o2.at[rows, pl.ds(D, D)])
            pltpu.sync_copy(nj_b.at[4], o2.at[rows, pl.ds(2 * D, D)])

    return k(users, pos, negf, u_true_s, u3, v)


def _tc_loss(p1, p2, das2, wt, dab):
    B = p1.shape[0]
    D = dab.shape[1]
    DAP = dab.shape[0]
    BLK = 1024
    grid = B // BLK

    def body(p1_ref, p2_ref, das_ref, w_ref, dab_ref, out_ref):
        x1 = p1_ref[...]
        x2 = p2_ref[...]
        onehot = (
            lax.broadcasted_iota(jnp.int32, (BLK, DAP), 1) == das_ref[...]
        ).astype(jnp.float32)
        u = jnp.dot(x1[:, 0:D], w_ref[...],
                    preferred_element_type=jnp.float32)
        u = u + jnp.dot(onehot, dab_ref[...],
                        preferred_element_type=jnp.float32)
        dpos = u - x1[:, D:2 * D] + _EPS
        dp = jnp.sqrt(jnp.sum(dpos * dpos, axis=1))
        negs = [x1[:, 2 * D:3 * D], x1[:, 3 * D:4 * D],
                x2[:, 0:D], x2[:, D:2 * D], x2[:, 2 * D:3 * D]]
        acc = jnp.zeros((), jnp.float32)
        for xj in negs:
            dneg = u - xj + _EPS
            dn = jnp.sqrt(jnp.sum(dneg * dneg, axis=1))
            acc = acc + jnp.sum(jnp.maximum(dp - dn + 1.0, 0.0))

        @pl.when(pl.program_id(0) == 0)
        def _():
            out_ref[...] = jnp.zeros_like(out_ref)

        out_ref[...] += (acc * (1.0 / B)).reshape(1, 1)

    out = pl.pallas_call(
        body,
        grid=(grid,),
        in_specs=[
            pl.BlockSpec((BLK, _PK), lambda i: (i, 0)),
            pl.BlockSpec((BLK, _PK), lambda i: (i, 0)),
            pl.BlockSpec((BLK, 1), lambda i: (i, 0)),
            pl.BlockSpec((D, D), lambda i: (0, 0)),
            pl.BlockSpec((DAP, D), lambda i: (0, 0)),
        ],
        out_specs=pl.BlockSpec((1, 1), lambda i: (0, 0)),
        out_shape=jax.ShapeDtypeStruct((1, 1), jnp.float32),
    )(p1, p2, das2, wt, dab)
    return out[0, 0]


def kernel(phase, users, pos_job_ids, behavior_ids, das, neg_job_id_lists,
           U_true, U, V, da_tab, W, b):
    del phase, behavior_ids
    NUSER, BEHm1, D = U.shape
    B = users.shape[0]
    DA = da_tab.shape[0] - 1
    u_true_s = U_true[:NUSER]
    u2 = U.reshape(NUSER, BEHm1 * D)
    negf = neg_job_id_lists.T.reshape(-1)
    das_c = jnp.clip(das, 0, DA).astype(jnp.int32).reshape(B, 1)
    DAP = 128
    dab = jnp.zeros((DAP, D), jnp.float32).at[:DA + 1].set(da_tab + b[None, :])
    p1, p2 = _sc_gather(users, pos_job_ids, negf, u_true_s, u2, V)
    return _tc_loss(p1, p2, das_c, W.T, dab)
